# baseline (speedup 1.0000x reference)
import functools

import jax
import jax.numpy as jnp
from jax import lax
from jax.experimental import pallas as pl
from jax.experimental.pallas import tpu as pltpu
from jax.experimental.pallas import tpu_sc as plsc

_CHUNK = 128
_L = 16


@functools.lru_cache(maxsize=None)
def _build(B, CV, D, nc, ns):
    R = 128 // D
    assert R == 8 and D == 16
    nw = nc * ns
    b_per_w = B // nw
    nchunk = b_per_w // _CHUNK
    orows = b_per_w // R
    mesh = plsc.VectorSubcoreMesh(core_axis_name="c", subcore_axis_name="s")

    @functools.partial(
        pl.kernel,
        mesh=mesh,
        out_type=jax.ShapeDtypeStruct((B // R, 128), jnp.float32),
        scratch_types=[
            pltpu.VMEM((b_per_w,), jnp.int32),
            pltpu.VMEM((nchunk, _CHUNK), jnp.int32),
            pltpu.VMEM((b_per_w, 128), jnp.float32),
            pltpu.VMEM((orows, 128), jnp.float32),
            pltpu.SemaphoreType.DMA,
        ],
        compiler_params=pltpu.CompilerParams(
            use_tc_tiling_on_sc=True, needs_layout_passes=False
        ),
    )
    def gather_kernel(idx_hbm, table_hbm, out_hbm, idx_v, cidx_v, comp_v,
                      out_v, sem):
        wid = lax.axis_index("s") * nc + lax.axis_index("c")
        pltpu.sync_copy(idx_hbm.at[wid], idx_v)

        for t in range(b_per_w // _L):
            iv = idx_v[pl.ds(t * _L, _L)]
            c, pos = divmod(t * _L, _CHUNK)
            cidx_v[c, pl.ds(pos, _L)] = iv >> 3

        copies = [
            pltpu.async_copy(
                table_hbm.at[cidx_v.at[c]],
                comp_v.at[pl.ds(c * _CHUNK, _CHUNK)],
                sem,
            )
            for c in range(nchunk)
        ]
        for cp in copies:
            cp.wait()

        iota = lax.iota(jnp.int32, _L)
        orow_base = iota >> 3
        ocol_base = (iota & 7) * D
        for g in range(b_per_w // _L):
            iv = idx_v[pl.ds(g * _L, _L)]
            col_base = (iv & 7) * D
            rowv = g * _L + iota
            orow = g * (_L // R) + orow_base
            for d in range(D):
                vals = plsc.load_gather(comp_v, [rowv, col_base + d])
                plsc.store_scatter(out_v, [orow, ocol_base + d], vals)

        pltpu.sync_copy(out_v, out_hbm.at[pl.ds(wid * orows, orows)])

    return gather_kernel


def kernel(idx, context_hat):
    B = idx.shape[0]
    V, D = context_hat.shape
    info = plsc.get_sparse_core_info()
    nc, ns = info.num_cores, info.num_subcores
    nw = nc * ns
    R = 128 // D
    idx_2d = idx.reshape(B).astype(jnp.int32).reshape(nw, B // nw)
    table_c = context_hat.reshape(V // R, 128)
    out_c = _build(B, V // R, D, nc, ns)(idx_2d, table_c)
    return out_c.reshape(B, D)


# speedup vs baseline: 5.4286x; 5.4286x over previous
import functools

import jax
import jax.numpy as jnp
from jax import lax
from jax.experimental import pallas as pl
from jax.experimental.pallas import tpu as pltpu
from jax.experimental.pallas import tpu_sc as plsc

_L = 16


@functools.lru_cache(maxsize=None)
def _build(B, V, D, nc, ns):
    nw = nc * ns
    b_per_w = B // nw
    groups = b_per_w // _L
    mesh = plsc.VectorSubcoreMesh(core_axis_name="c", subcore_axis_name="s")

    @functools.partial(
        pl.kernel,
        mesh=mesh,
        out_type=jax.ShapeDtypeStruct((D, B), jnp.float32),
        scratch_types=[
            pltpu.VMEM((b_per_w,), jnp.int32),
            pltpu.VMEM((_L, D, 128), jnp.float32),
            pltpu.VMEM((D, b_per_w), jnp.float32),
            [pltpu.SemaphoreType.DMA for _ in range(_L)],
        ],
        compiler_params=pltpu.CompilerParams(
            use_tc_tiling_on_sc=True, needs_layout_passes=False
        ),
    )
    def gather_kernel(idx_hbm, table_hbm, out_hbm, idx_v, blk_v, out_v, sems):
        wid = lax.axis_index("s") * nc + lax.axis_index("c")
        pltpu.sync_copy(idx_hbm.at[wid], idx_v)
        iota = lax.iota(jnp.int32, _L)

        def fetch(slot, i):
            off = pl.multiple_of((i >> 7) * 128, 128)
            pltpu.async_copy(
                table_hbm.at[:, pl.ds(off, 128)], blk_v.at[slot], sems[slot]
            )

        def extract(slot, i, r):
            pltpu.make_async_copy(
                table_hbm.at[:, pl.ds(0, 128)], blk_v.at[slot], sems[slot]
            ).wait()
            word = jnp.full((_L,), i & 127, jnp.int32)
            vals = plsc.load_gather(blk_v.at[slot], [iota, word])
            plsc.store_scatter(out_v, [iota, jnp.full((_L,), r, jnp.int32)], vals)

        iv0 = idx_v[pl.ds(0, _L)]
        for k in range(_L):
            fetch(k, iv0[k])

        def round_body(g, iv_cur):
            iv_next = idx_v[pl.ds((g + 1) * _L, _L)]
            for k in range(_L):
                extract(k, iv_cur[k], g * _L + k)
                fetch(k, iv_next[k])
            return iv_next

        iv_last = lax.fori_loop(0, groups - 1, round_body, iv0)
        for k in range(_L):
            extract(k, iv_last[k], (groups - 1) * _L + k)

        base = pl.multiple_of(wid * b_per_w, 128)
        pltpu.sync_copy(out_v, out_hbm.at[:, pl.ds(base, b_per_w)])

    return gather_kernel


def kernel(idx, context_hat):
    B = idx.shape[0]
    V, D = context_hat.shape
    info = plsc.get_sparse_core_info()
    nc, ns = info.num_cores, info.num_subcores
    nw = nc * ns
    idx_2d = idx.reshape(B).astype(jnp.int32).reshape(nw, B // nw)
    out_t = _build(B, V, D, nc, ns)(idx_2d, context_hat.T)
    return out_t.T
